# baseline (speedup 1.0000x reference)
import functools

import jax
import jax.numpy as jnp
from jax import lax
from jax.experimental import pallas as pl
from jax.experimental.pallas import tpu as pltpu
from jax.experimental.pallas import tpu_sc as plsc

NUM_SPACE = 256
MAX_TIME = 16
EMBED_DIM = 768

NC = 2
NS = 16
NW = NC * NS

BATCH = 4
NUM_PATCHES = 4096
ROWS_PER_W = NUM_PATCHES // NW
LANES = 16
COLS = EMBED_DIM // LANES


CHUNK = 32
NCHUNK = ROWS_PER_W // CHUNK


def _sc_body(space_hbm, time_hbm, out_hbm, block_v, time_v, rsem, wsem):
    c = lax.axis_index("c")
    s = lax.axis_index("s")
    wid = s * NC + c
    row0 = wid * ROWS_PER_W
    t_idx = wid // (NUM_SPACE // ROWS_PER_W)
    s_off = (wid % (NUM_SPACE // ROWS_PER_W)) * ROWS_PER_W

    tread = pltpu.async_copy(time_hbm.at[pl.ds(t_idx, 1)], time_v, rsem)
    reads = [
        pltpu.async_copy(
            space_hbm.at[pl.ds(s_off + ch * CHUNK, CHUNK)],
            block_v.at[pl.ds(ch * CHUNK, CHUNK)],
            rsem,
        )
        for ch in range(NCHUNK)
    ]
    tread.wait()

    tvals = tuple(time_v[0, pl.ds(col * LANES, LANES)] for col in range(COLS))

    def row_body(r, carry):
        for col in range(COLS):
            sl = pl.ds(col * LANES, LANES)
            block_v[r, sl] = block_v[r, sl] + carry[col]
        return carry

    handles = []
    for ch in range(NCHUNK):
        reads[ch].wait()
        lax.fori_loop(ch * CHUNK, (ch + 1) * CHUNK, row_body, tvals)
        src = block_v.at[pl.ds(ch * CHUNK, CHUNK)]
        for b in range(BATCH):
            handles.append(
                pltpu.async_copy(
                    src, out_hbm.at[b, pl.ds(row0 + ch * CHUNK, CHUNK)], wsem
                )
            )
    for h in handles:
        h.wait()


def kernel(x, space_embedding, time_embedding):
    batch, num_patches, dim = x.shape
    mesh = plsc.VectorSubcoreMesh(core_axis_name="c", subcore_axis_name="s")
    run = functools.partial(
        pl.kernel,
        mesh=mesh,
        out_type=jax.ShapeDtypeStruct((batch, num_patches, dim), jnp.float32),
        scratch_types=[
            pltpu.VMEM((ROWS_PER_W, EMBED_DIM), jnp.float32),
            pltpu.VMEM((1, EMBED_DIM), jnp.float32),
            pltpu.SemaphoreType.DMA,
            pltpu.SemaphoreType.DMA,
        ],
    )(_sc_body)
    return run(space_embedding, time_embedding)


# speedup vs baseline: 1.2561x; 1.2561x over previous
import functools

import jax
import jax.numpy as jnp
from jax import lax
from jax.experimental import pallas as pl
from jax.experimental.pallas import tpu as pltpu
from jax.experimental.pallas import tpu_sc as plsc

NUM_SPACE = 256
MAX_TIME = 16
EMBED_DIM = 768

NC = 2
NS = 16
NW = NC * NS

BATCH = 4
NUM_PATCHES = 4096
ROWS_PER_W = NUM_PATCHES // NW
LANES = 16
COLS = EMBED_DIM // LANES


CHUNK = 32
NCHUNK = ROWS_PER_W // CHUNK


STRIPE = ROWS_PER_W // NS


def _sc_body(space_hbm, time_hbm, out_hbm, block_v, time_v, space_sh, rsem, wsem):
    c = lax.axis_index("c")
    s = lax.axis_index("s")
    wid = s * NC + c
    row0 = wid * ROWS_PER_W
    t_idx = wid // (NUM_SPACE // ROWS_PER_W)
    s_off = (wid % (NUM_SPACE // ROWS_PER_W)) * ROWS_PER_W

    tread = pltpu.async_copy(time_hbm.at[pl.ds(t_idx, 1)], time_v, rsem)
    stripe = pltpu.async_copy(
        space_hbm.at[pl.ds(s_off + s * STRIPE, STRIPE)],
        space_sh.at[pl.ds(s * STRIPE, STRIPE)],
        rsem,
    )
    stripe.wait()
    plsc.subcore_barrier()
    reads = [
        pltpu.async_copy(
            space_sh.at[pl.ds(ch * CHUNK, CHUNK)],
            block_v.at[pl.ds(ch * CHUNK, CHUNK)],
            rsem,
        )
        for ch in range(NCHUNK)
    ]
    tread.wait()

    tvals = tuple(time_v[0, pl.ds(col * LANES, LANES)] for col in range(COLS))

    def row_body(r, carry):
        for col in range(COLS):
            sl = pl.ds(col * LANES, LANES)
            block_v[r, sl] = block_v[r, sl] + carry[col]
        return carry

    handles = []
    for ch in range(NCHUNK):
        reads[ch].wait()
        lax.fori_loop(ch * CHUNK, (ch + 1) * CHUNK, row_body, tvals)
        src = block_v.at[pl.ds(ch * CHUNK, CHUNK)]
        for b in range(BATCH):
            handles.append(
                pltpu.async_copy(
                    src, out_hbm.at[b, pl.ds(row0 + ch * CHUNK, CHUNK)], wsem
                )
            )
    for h in handles:
        h.wait()


def kernel(x, space_embedding, time_embedding):
    batch, num_patches, dim = x.shape
    mesh = plsc.VectorSubcoreMesh(core_axis_name="c", subcore_axis_name="s")
    run = functools.partial(
        pl.kernel,
        mesh=mesh,
        out_type=jax.ShapeDtypeStruct((batch, num_patches, dim), jnp.float32),
        scratch_types=[
            pltpu.VMEM((ROWS_PER_W, EMBED_DIM), jnp.float32),
            pltpu.VMEM((1, EMBED_DIM), jnp.float32),
            pltpu.VMEM_SHARED((ROWS_PER_W, EMBED_DIM), jnp.float32),
            pltpu.SemaphoreType.DMA,
            pltpu.SemaphoreType.DMA,
        ],
    )(_sc_body)
    return run(space_embedding, time_embedding)
